# baseline (speedup 1.0000x reference)
import functools

import jax
import jax.numpy as jnp
from jax import lax
from jax.experimental import pallas as pl
from jax.experimental.pallas import tpu as pltpu
from jax.experimental.pallas import tpu_sc as plsc

_VOCAB = 1000
_B = 1024
_T = 50
_NW = 32
_ROWS_W = _B // _NW


def _make_gather():
    mesh = plsc.VectorSubcoreMesh(core_axis_name="c", subcore_axis_name="s")

    @functools.partial(
        pl.kernel,
        mesh=mesh,
        compiler_params=pltpu.CompilerParams(use_tc_tiling_on_sc=False),
        out_type=jax.ShapeDtypeStruct((_B, _T, _VOCAB), jnp.float32),
        scratch_types=[
            pltpu.VMEM_SHARED((_VOCAB, _VOCAB), jnp.float32),
            pltpu.VMEM((_ROWS_W, _T), jnp.int32),
            pltpu.VMEM((1, _T, _VOCAB), jnp.float32),
            pltpu.SemaphoreType.DMA,
            pltpu.SemaphoreType.DMA,
        ],
    )
    def gather_kernel(idx_hbm, w_hbm, out_hbm, w_sh, idx_v, buf, sg, ss):
        sid = lax.axis_index("s")
        wid = sid * 2 + lax.axis_index("c")
        base = wid * _ROWS_W

        @pl.when(sid == 0)
        def _():
            pltpu.sync_copy(w_hbm, w_sh)

        pltpu.sync_copy(idx_hbm.at[pl.ds(base, _ROWS_W)], idx_v)
        plsc.subcore_barrier()

        def body(r, _):
            pltpu.async_copy(w_sh.at[idx_v.at[r]], buf.at[0], sg).wait()
            pltpu.async_copy(
                buf, out_hbm.at[pl.ds(base + r, 1)], ss).wait()
            return 0

        lax.fori_loop(0, _ROWS_W, body, 0)

    return gather_kernel


_gather = _make_gather()


def kernel(idx, W):
    return _gather(idx.astype(jnp.int32), W)
